# baseline (speedup 1.0000x reference)
import jax
import jax.numpy as jnp
import numpy as np
from jax import lax
from jax.experimental import pallas as pl
from jax.experimental.pallas import tpu as pltpu
from jax.experimental.pallas import tpu_sc as plsc

B, C, D, H, W = 2, 128, 32, 64, 64
DHW = D * H * W
HW = H * W
PD = PH = PW = 7
CELLS = PD * PH * PW
STRIDE = 344
NR = 1000
NR_PAD = 1024
NR_STAGE = 1056
NS = 16
ROIS_C0 = 48
ROIS_C1 = 16
GROUPS_MAX = ROIS_C0 // 16

_LIN_T = [np.float32(np.float32(k) / np.float32(6)) for k in range(6)]
_LIN_OMT = [np.float32(np.float32(1) - t) for t in _LIN_T]


def _grid_indices(a, b, hi):
    af = jnp.clip(a, 0.0, float(hi - 1))
    bf = jnp.clip(b, 0.0, float(hi - 1))
    out = []
    for k in range(7):
        if k == 6:
            g = bf
        else:
            g = af * _LIN_OMT[k] + bf * _LIN_T[k]
        out.append(jnp.clip(g.astype(jnp.int32), 0, hi - 1))
    return out


def _roi_gather_body(table, rois, out, roi_v, idx_v, buf0, buf1,
                     gsem0, gsem1, wsem0, wsem1):
    s = lax.axis_index("s")
    c = lax.axis_index("c")
    base_roi = jnp.where(c == 0, s * ROIS_C0, NR_PAD - NS * ROIS_C1 + s * ROIS_C1)
    my_rois = jnp.where(c == 0, ROIS_C0, ROIS_C1)

    pltpu.sync_copy(rois.at[pl.ds(base_roi * 8, ROIS_C0 * 8)], roi_v)

    lanes = lax.iota(jnp.int32, 16)
    zeros = jnp.zeros((16,), jnp.int32)
    for g in range(GROUPS_MAX):

        @pl.when(jnp.logical_or(c == 0, g == 0))
        def _():
            base_addr = g * 128 + lanes * 8

            def fld(f):
                return plsc.load_gather(roi_v, [base_addr + f])

            bi = jnp.clip(fld(0).astype(jnp.int32), 0, B - 1)
            x1, y1, z1 = fld(1), fld(2), fld(3)
            x2, y2, z2 = fld(4), fld(5), fld(6)
            ix = _grid_indices(x1, x2, W)
            iy = _grid_indices(y1, y2, H)
            iz = _grid_indices(z1, z2, D)

            rowb = bi * DHW
            pos_base = (g * 16 + lanes) * STRIDE
            cell = 0
            for k3 in range(PD):
                t3 = rowb + iz[k3] * HW
                for k2 in range(PH):
                    t32 = t3 + iy[k2] * W
                    for k1 in range(PW):
                        plsc.store_scatter(idx_v, [pos_base + cell], t32 + ix[k1])
                        cell += 1
            plsc.store_scatter(idx_v, [pos_base + CELLS], zeros)

    bufs = (buf0, buf1)
    gsems = (gsem0, gsem1)
    wsems = (wsem0, wsem1)
    chunk_off = (0, 112, 224, 336)
    chunk_len = (112, 112, 112, 8)

    def roi_pair(i, _):
        for b in range(2):
            l = 2 * i + b
            buf, gsem, wsem = bufs[b], gsems[b], wsems[b]

            @pl.when(i >= 1)
            def _():
                pltpu.make_async_copy(
                    buf, out.at[pl.ds(0, STRIDE)], wsem).wait()

            cps = []
            for o, n in zip(chunk_off, chunk_len):
                cps.append(pltpu.async_copy(
                    table.at[idx_v.at[pl.ds(l * STRIDE + o, n)]],
                    buf.at[pl.ds(o, n)], gsem))
            for cp in cps:
                cp.wait()
            pltpu.async_copy(
                buf, out.at[pl.ds((base_roi + l) * STRIDE, STRIDE)], wsem)
        return _

    lax.fori_loop(0, my_rois // 2, roi_pair, None)
    for b in range(2):
        pltpu.make_async_copy(bufs[b], out.at[pl.ds(0, STRIDE)], wsems[b]).wait()


_mesh = plsc.VectorSubcoreMesh(core_axis_name="c", subcore_axis_name="s")

_roi_gather = pl.kernel(
    _roi_gather_body,
    out_type=jax.ShapeDtypeStruct((NR_PAD * STRIDE, C), jnp.float32),
    mesh=_mesh,
    scratch_types=[
        pltpu.VMEM((ROIS_C0 * 8,), jnp.float32),
        pltpu.VMEM((ROIS_C0 * STRIDE,), jnp.int32),
        pltpu.VMEM((STRIDE, C), jnp.float32),
        pltpu.VMEM((STRIDE, C), jnp.float32),
        pltpu.SemaphoreType.DMA,
        pltpu.SemaphoreType.DMA,
        pltpu.SemaphoreType.DMA,
        pltpu.SemaphoreType.DMA,
    ],
    compiler_params=pltpu.CompilerParams(needs_layout_passes=False),
)


_YB = 8


def _to_channel_last_body(f_ref, t_ref):
    for j in range(_YB):
        t_ref[pl.ds(j * W, W), :] = jnp.transpose(f_ref[0, :, 0, j, :], (1, 0))


_to_channel_last = pl.pallas_call(
    _to_channel_last_body,
    grid=(B, D, H // _YB),
    in_specs=[pl.BlockSpec((1, C, 1, _YB, W), lambda b, z, h: (b, 0, z, h, 0))],
    out_specs=pl.BlockSpec(
        (_YB * W, C), lambda b, z, h: ((b * D + z) * (H // _YB) + h, 0)),
    out_shape=jax.ShapeDtypeStruct((B * DHW, C), jnp.float32),
)


_RB = 4


def _to_channel_major_body(g_ref, o_ref):
    v = g_ref[...].reshape(_RB, STRIDE, C)
    o_ref[...] = jnp.transpose(v, (0, 2, 1))[:, :, :CELLS]


_to_channel_major = pl.pallas_call(
    _to_channel_major_body,
    grid=(NR // _RB,),
    in_specs=[pl.BlockSpec((_RB * STRIDE, C), lambda i: (i, 0))],
    out_specs=pl.BlockSpec((_RB, C, CELLS), lambda i: (i, 0, 0)),
    out_shape=jax.ShapeDtypeStruct((NR, C, CELLS), jnp.float32),
)


@jax.jit
def kernel(features, rois):
    table = _to_channel_last(features)
    rois_p = jnp.pad(rois, ((0, NR_STAGE - NR), (0, 1))).reshape(-1)
    gathered = _roi_gather(table, rois_p)
    pooled = _to_channel_major(gathered)
    return pooled.reshape(NR, C, PD, PH, PW)


# speedup vs baseline: 7.3521x; 7.3521x over previous
import jax
import jax.numpy as jnp
import numpy as np
from jax import lax
from jax.experimental import pallas as pl
from jax.experimental.pallas import tpu as pltpu
from jax.experimental.pallas import tpu_sc as plsc

B, C, D, H, W = 2, 128, 32, 64, 64
DHW = D * H * W
HW = H * W
PD = PH = PW = 7
CELLS = PD * PH * PW
NR = 1000
NR_PAD = 1024
NGRP_FULL = NR // 16
CELLS_MAX = 22
CH = 40
NBUF = 4

_LIN_T = [np.float32(np.float32(k) / np.float32(6)) for k in range(6)]
_LIN_OMT = [np.float32(np.float32(1) - t) for t in _LIN_T]


def _grid_indices(a, b, hi):
    af = jnp.clip(a, 0.0, float(hi - 1))
    bf = jnp.clip(b, 0.0, float(hi - 1))
    out = []
    for k in range(7):
        if k == 6:
            g = bf
        else:
            g = af * _LIN_OMT[k] + bf * _LIN_T[k]
        out.append(jnp.clip(g.astype(jnp.int32), 0, hi - 1))
    return out


def _group_contribs(roi_v, g, lanes):
    base_addr = g * 128 + lanes * 8

    def fld(f):
        return plsc.load_gather(roi_v, [base_addr + f])

    bi = jnp.clip(fld(0).astype(jnp.int32), 0, B - 1)
    x1, y1, z1 = fld(1), fld(2), fld(3)
    x2, y2, z2 = fld(4), fld(5), fld(6)
    ix = _grid_indices(x1, x2, W)
    iy = _grid_indices(y1, y2, H)
    iz = _grid_indices(z1, z2, D)
    rowb = bi * DHW
    zc = [rowb + v * HW for v in iz]
    yc = [v * W for v in iy]
    return zc + yc + ix


def _roi_gather_body(table, rois, out, roi_v, tab_v, idx_v, bufs, gsems, wsems):
    s = lax.axis_index("s")
    c = lax.axis_index("c")
    base_cell = jnp.where(s < 7, s * 22, 21 * s + 7)
    ncells = jnp.where(s < 7, 22, 21)
    nch = jnp.where(c == 0, ncells * (1000 // CH), 0)

    lanes = lax.iota(jnp.int32, 16)

    @pl.when(c == 0)
    def _():
        pltpu.sync_copy(rois, roi_v)

        def phase1(g, carry):
            vecs = _group_contribs(roi_v, g, lanes)
            for k in range(21):
                tab_v[pl.ds(k * 1000 + g * 16, 16)] = vecs[k]
            return carry

        lax.fori_loop(0, NGRP_FULL, phase1, None)
        vecs = _group_contribs(roi_v, NGRP_FULL, lanes)
        tail_mask = lanes < 8
        tail_pos = NGRP_FULL * 16 + lanes
        for k in range(21):
            plsc.store_scatter(tab_v, [k * 1000 + tail_pos], vecs[k],
                               mask=tail_mask)

        addr = []
        for jj in range(CELLS_MAX):
            j = base_cell + jj
            k3 = j // 49
            k2 = (j // 7) % 7
            k1 = j % 7
            addr.append((k3 * 1000, (7 + k2) * 1000, (14 + k1) * 1000))

        def phase2(g, carry):
            g16 = g * 16
            for jj in range(CELLS_MAX):
                az, ay, ax = addr[jj]
                v = (tab_v[pl.ds(az + g16, 16)]
                     + tab_v[pl.ds(ay + g16, 16)]
                     + tab_v[pl.ds(ax + g16, 16)])
                idx_v[pl.ds(jj * 1000 + g16, 16)] = v
            return carry

        lax.fori_loop(0, NGRP_FULL, phase2, None)
        tp = NGRP_FULL * 16
        for jj in range(CELLS_MAX):
            az, ay, ax = addr[jj]
            v = (tab_v[pl.ds(az + tp, 16)]
                 + tab_v[pl.ds(ay + tp, 16)]
                 + tab_v[pl.ds(ax + tp, 16)])
            plsc.store_scatter(idx_v, [jj * 1000 + tp + lanes], v,
                               mask=tail_mask)

    out_base = base_cell * 1000

    def ring(r, _):
        for b in range(NBUF):
            t = NBUF * r + b

            @pl.when(t < nch)
            def _():
                @pl.when(t >= NBUF)
                def _():
                    pltpu.make_async_copy(
                        bufs[b], out.at[pl.ds(0, CH)], wsems[b]).wait()

                pltpu.async_copy(
                    table.at[idx_v.at[pl.ds(t * CH, CH)]], bufs[b], gsems[b])

            t2 = t - (NBUF - 1)
            b2 = (b + 1) % NBUF

            @pl.when(jnp.logical_and(t2 >= 0, t2 < nch))
            def _():
                pltpu.make_async_copy(
                    table.at[idx_v.at[pl.ds(0, CH)]], bufs[b2], gsems[b2]).wait()
                pltpu.async_copy(
                    bufs[b2], out.at[pl.ds(out_base + t2 * CH, CH)], wsems[b2])
        return _

    lax.fori_loop(0, (nch + 2 * (NBUF - 1)) // NBUF, ring, None)
    for b in range(NBUF):
        @pl.when(nch > b)
        def _():
            pltpu.make_async_copy(bufs[b], out.at[pl.ds(0, CH)], wsems[b]).wait()


def _body(table, rois, out, roi_v, tab_v, idx_v, b0, b1, b2, b3,
          g0, g1, g2, g3, w0, w1, w2, w3):
    _roi_gather_body(table, rois, out, roi_v, tab_v, idx_v,
                     (b0, b1, b2, b3), (g0, g1, g2, g3), (w0, w1, w2, w3))


_mesh = plsc.VectorSubcoreMesh(core_axis_name="c", subcore_axis_name="s")

_roi_gather = pl.kernel(
    _body,
    out_type=jax.ShapeDtypeStruct((CELLS * NR, C), jnp.float32),
    mesh=_mesh,
    scratch_types=[
        pltpu.VMEM((NR_PAD * 8,), jnp.float32),
        pltpu.VMEM((21 * 1000 + 8, ), jnp.int32),
        pltpu.VMEM((CELLS_MAX * 1000,), jnp.int32),
    ] + [pltpu.VMEM((CH, C), jnp.float32)] * NBUF
      + [pltpu.SemaphoreType.DMA] * (2 * NBUF),
    compiler_params=pltpu.CompilerParams(needs_layout_passes=False),
)


@jax.jit
def kernel(features, rois):
    table = jnp.transpose(features, (0, 2, 3, 4, 1)).reshape(B * DHW, C)
    rois_p = jnp.pad(rois, ((0, NR_PAD - NR), (0, 1))).reshape(-1)
    gathered = _roi_gather(table, rois_p)
    pooled = gathered.reshape(PD, PH, PW, NR, C)
    return jnp.transpose(pooled, (3, 4, 0, 1, 2))


# speedup vs baseline: 9.0662x; 1.2332x over previous
import jax
import jax.numpy as jnp
import numpy as np
from jax import lax
from jax.experimental import pallas as pl
from jax.experimental.pallas import tpu as pltpu
from jax.experimental.pallas import tpu_sc as plsc

B, C, D, H, W = 2, 128, 32, 64, 64
DHW = D * H * W
HW = H * W
PD = PH = PW = 7
CELLS = PD * PH * PW
NR = 1000
NR_PAD = 1024
NGRP_FULL = NR // 16
CELLS_MAX = 22
CH = 40
NBUF = 6

_LIN_T = [np.float32(np.float32(k) / np.float32(6)) for k in range(6)]
_LIN_OMT = [np.float32(np.float32(1) - t) for t in _LIN_T]


def _grid_indices(a, b, hi):
    af = jnp.clip(a, 0.0, float(hi - 1))
    bf = jnp.clip(b, 0.0, float(hi - 1))
    out = []
    for k in range(7):
        if k == 6:
            g = bf
        else:
            g = af * _LIN_OMT[k] + bf * _LIN_T[k]
        out.append(jnp.clip(g.astype(jnp.int32), 0, hi - 1))
    return out


def _group_contribs(roi_v, g, lanes):
    base_addr = g * 128 + lanes * 8

    def fld(f):
        return plsc.load_gather(roi_v, [base_addr + f])

    bi = jnp.clip(fld(0).astype(jnp.int32), 0, B - 1)
    x1, y1, z1 = fld(1), fld(2), fld(3)
    x2, y2, z2 = fld(4), fld(5), fld(6)
    ix = _grid_indices(x1, x2, W)
    iy = _grid_indices(y1, y2, H)
    iz = _grid_indices(z1, z2, D)
    rowb = bi * DHW
    zc = [rowb + v * HW for v in iz]
    yc = [v * W for v in iy]
    return zc + yc + ix


def _roi_gather_body(table, rois, out, roi_v, tab_v, idx_v, bufs, gsems, wsems):
    s = lax.axis_index("s")
    c = lax.axis_index("c")
    base_cell = jnp.where(s < 7, s * 22, 21 * s + 7)
    ncells = jnp.where(s < 7, 22, 21)
    nch = jnp.where(c == 0, ncells * (1000 // CH), 0)

    lanes = lax.iota(jnp.int32, 16)

    @pl.when(c == 0)
    def _():
        pltpu.sync_copy(rois, roi_v)

        def phase1(g, carry):
            vecs = _group_contribs(roi_v, g, lanes)
            for k in range(21):
                tab_v[pl.ds(k * 1000 + g * 16, 16)] = vecs[k]
            return carry

        lax.fori_loop(0, NGRP_FULL, phase1, None)
        vecs = _group_contribs(roi_v, NGRP_FULL, lanes)
        tail_mask = lanes < 8
        tail_pos = NGRP_FULL * 16 + lanes
        for k in range(21):
            plsc.store_scatter(tab_v, [k * 1000 + tail_pos], vecs[k],
                               mask=tail_mask)

        addr = []
        for jj in range(CELLS_MAX):
            j = base_cell + jj
            k3 = j // 49
            k2 = (j // 7) % 7
            k1 = j % 7
            addr.append((k3 * 1000, (7 + k2) * 1000, (14 + k1) * 1000))

        def phase2(g, carry):
            g16 = g * 16
            for jj in range(CELLS_MAX):
                az, ay, ax = addr[jj]
                v = (tab_v[pl.ds(az + g16, 16)]
                     + tab_v[pl.ds(ay + g16, 16)]
                     + tab_v[pl.ds(ax + g16, 16)])
                idx_v[pl.ds(jj * 1000 + g16, 16)] = v
            return carry

        lax.fori_loop(0, NGRP_FULL, phase2, None)
        tp = NGRP_FULL * 16
        for jj in range(CELLS_MAX):
            az, ay, ax = addr[jj]
            v = (tab_v[pl.ds(az + tp, 16)]
                 + tab_v[pl.ds(ay + tp, 16)]
                 + tab_v[pl.ds(ax + tp, 16)])
            plsc.store_scatter(idx_v, [jj * 1000 + tp + lanes], v,
                               mask=tail_mask)

    out_base = base_cell * 1000

    def ring(r, _):
        for b in range(NBUF):
            t = NBUF * r + b

            @pl.when(t < nch)
            def _():
                @pl.when(t >= NBUF)
                def _():
                    pltpu.make_async_copy(
                        bufs[b], out.at[pl.ds(0, CH)], wsems[b]).wait()

                pltpu.async_copy(
                    table.at[idx_v.at[pl.ds(t * CH, CH)]], bufs[b], gsems[b])

            t2 = t - (NBUF - 1)
            b2 = (b + 1) % NBUF

            @pl.when(jnp.logical_and(t2 >= 0, t2 < nch))
            def _():
                pltpu.make_async_copy(
                    table.at[idx_v.at[pl.ds(0, CH)]], bufs[b2], gsems[b2]).wait()
                pltpu.async_copy(
                    bufs[b2], out.at[pl.ds(out_base + t2 * CH, CH)], wsems[b2])
        return _

    lax.fori_loop(0, (nch + 2 * (NBUF - 1)) // NBUF, ring, None)
    for b in range(NBUF):
        @pl.when(nch > b)
        def _():
            pltpu.make_async_copy(bufs[b], out.at[pl.ds(0, CH)], wsems[b]).wait()


def _body(table, rois, out, roi_v, tab_v, idx_v, *rest):
    _roi_gather_body(table, rois, out, roi_v, tab_v, idx_v,
                     rest[:NBUF], rest[NBUF:2 * NBUF], rest[2 * NBUF:])


_mesh = plsc.VectorSubcoreMesh(core_axis_name="c", subcore_axis_name="s")

_roi_gather = pl.kernel(
    _body,
    out_type=jax.ShapeDtypeStruct((CELLS * NR, C), jnp.float32),
    mesh=_mesh,
    scratch_types=[
        pltpu.VMEM((NR_PAD * 8,), jnp.float32),
        pltpu.VMEM((21 * 1000 + 8, ), jnp.int32),
        pltpu.VMEM((CELLS_MAX * 1000,), jnp.int32),
    ] + [pltpu.VMEM((CH, C), jnp.float32)] * NBUF
      + [pltpu.SemaphoreType.DMA] * (2 * NBUF),
    compiler_params=pltpu.CompilerParams(needs_layout_passes=False),
)


@jax.jit
def kernel(features, rois):
    table = jnp.transpose(features, (0, 2, 3, 4, 1)).reshape(B * DHW, C)
    rois_p = jnp.pad(rois, ((0, NR_PAD - NR), (0, 1))).reshape(-1)
    gathered = _roi_gather(table, rois_p)
    pooled = gathered.reshape(PD, PH, PW, NR, C)
    return jnp.transpose(pooled, (3, 4, 0, 1, 2))


# speedup vs baseline: 9.6970x; 1.0696x over previous
import jax
import jax.numpy as jnp
import numpy as np
from jax import lax
from jax.experimental import pallas as pl
from jax.experimental.pallas import tpu as pltpu
from jax.experimental.pallas import tpu_sc as plsc

B, C, D, H, W = 2, 128, 32, 64, 64
DHW = D * H * W
HW = H * W
PD = PH = PW = 7
CELLS = PD * PH * PW
NR = 1000
NR_PAD = 1024
NGRP_FULL = NR // 16
CELLS_MAX = 22
CH = 40
NBUF = 8

_LIN_T = [np.float32(np.float32(k) / np.float32(6)) for k in range(6)]
_LIN_OMT = [np.float32(np.float32(1) - t) for t in _LIN_T]


def _grid_indices(a, b, hi):
    af = jnp.clip(a, 0.0, float(hi - 1))
    bf = jnp.clip(b, 0.0, float(hi - 1))
    out = []
    for k in range(7):
        if k == 6:
            g = bf
        else:
            g = af * _LIN_OMT[k] + bf * _LIN_T[k]
        out.append(jnp.clip(g.astype(jnp.int32), 0, hi - 1))
    return out


def _group_contribs(roi_v, g, lanes):
    base_addr = g * 128 + lanes * 8

    def fld(f):
        return plsc.load_gather(roi_v, [base_addr + f])

    bi = jnp.clip(fld(0).astype(jnp.int32), 0, B - 1)
    x1, y1, z1 = fld(1), fld(2), fld(3)
    x2, y2, z2 = fld(4), fld(5), fld(6)
    ix = _grid_indices(x1, x2, W)
    iy = _grid_indices(y1, y2, H)
    iz = _grid_indices(z1, z2, D)
    rowb = bi * DHW
    zc = [rowb + v * HW for v in iz]
    yc = [v * W for v in iy]
    return zc + yc + ix


def _roi_gather_body(table, rois, out, roi_v, tab_v, idx_v, bufs, gsems, wsems):
    s = lax.axis_index("s")
    c = lax.axis_index("c")
    base_cell = jnp.where(s < 7, s * 22, 21 * s + 7)
    ncells = jnp.where(s < 7, 22, 21)
    nch = jnp.where(c == 0, ncells * (1000 // CH), 0)

    lanes = lax.iota(jnp.int32, 16)

    @pl.when(c == 0)
    def _():
        pltpu.sync_copy(rois, roi_v)

        def phase1(g, carry):
            vecs = _group_contribs(roi_v, g, lanes)
            for k in range(21):
                tab_v[pl.ds(k * 1000 + g * 16, 16)] = vecs[k]
            return carry

        lax.fori_loop(0, NGRP_FULL, phase1, None)
        vecs = _group_contribs(roi_v, NGRP_FULL, lanes)
        tail_mask = lanes < 8
        tail_pos = NGRP_FULL * 16 + lanes
        for k in range(21):
            plsc.store_scatter(tab_v, [k * 1000 + tail_pos], vecs[k],
                               mask=tail_mask)

        addr = []
        for jj in range(CELLS_MAX):
            j = base_cell + jj
            k3 = j // 49
            k2 = (j // 7) % 7
            k1 = j % 7
            addr.append((k3 * 1000, (7 + k2) * 1000, (14 + k1) * 1000))

        def phase2(g, carry):
            g16 = g * 16
            for jj in range(CELLS_MAX):
                az, ay, ax = addr[jj]
                v = (tab_v[pl.ds(az + g16, 16)]
                     + tab_v[pl.ds(ay + g16, 16)]
                     + tab_v[pl.ds(ax + g16, 16)])
                idx_v[pl.ds(jj * 1000 + g16, 16)] = v
            return carry

        lax.fori_loop(0, NGRP_FULL, phase2, None)
        tp = NGRP_FULL * 16
        for jj in range(CELLS_MAX):
            az, ay, ax = addr[jj]
            v = (tab_v[pl.ds(az + tp, 16)]
                 + tab_v[pl.ds(ay + tp, 16)]
                 + tab_v[pl.ds(ax + tp, 16)])
            plsc.store_scatter(idx_v, [jj * 1000 + tp + lanes], v,
                               mask=tail_mask)

    out_base = base_cell * 1000

    def ring(r, _):
        for b in range(NBUF):
            t = NBUF * r + b

            @pl.when(t < nch)
            def _():
                @pl.when(t >= NBUF)
                def _():
                    pltpu.make_async_copy(
                        bufs[b], out.at[pl.ds(0, CH)], wsems[b]).wait()

                pltpu.async_copy(
                    table.at[idx_v.at[pl.ds(t * CH, CH)]], bufs[b], gsems[b])

            t2 = t - (NBUF - 1)
            b2 = (b + 1) % NBUF

            @pl.when(jnp.logical_and(t2 >= 0, t2 < nch))
            def _():
                pltpu.make_async_copy(
                    table.at[idx_v.at[pl.ds(0, CH)]], bufs[b2], gsems[b2]).wait()
                pltpu.async_copy(
                    bufs[b2], out.at[pl.ds(out_base + t2 * CH, CH)], wsems[b2])
        return _

    lax.fori_loop(0, (nch + 2 * (NBUF - 1)) // NBUF, ring, None)
    for b in range(NBUF):
        @pl.when(nch > b)
        def _():
            pltpu.make_async_copy(bufs[b], out.at[pl.ds(0, CH)], wsems[b]).wait()


def _body(table, rois, out, roi_v, tab_v, idx_v, *rest):
    _roi_gather_body(table, rois, out, roi_v, tab_v, idx_v,
                     rest[:NBUF], rest[NBUF:2 * NBUF], rest[2 * NBUF:])


_mesh = plsc.VectorSubcoreMesh(core_axis_name="c", subcore_axis_name="s")

_roi_gather = pl.kernel(
    _body,
    out_type=jax.ShapeDtypeStruct((CELLS * NR, C), jnp.float32),
    mesh=_mesh,
    scratch_types=[
        pltpu.VMEM((NR_PAD * 8,), jnp.float32),
        pltpu.VMEM((21 * 1000 + 8, ), jnp.int32),
        pltpu.VMEM((CELLS_MAX * 1000,), jnp.int32),
    ] + [pltpu.VMEM((CH, C), jnp.float32)] * NBUF
      + [pltpu.SemaphoreType.DMA] * (2 * NBUF),
    compiler_params=pltpu.CompilerParams(needs_layout_passes=False),
)


@jax.jit
def kernel(features, rois):
    table = jnp.transpose(features, (0, 2, 3, 4, 1)).reshape(B * DHW, C)
    rois_p = jnp.pad(rois, ((0, NR_PAD - NR), (0, 1))).reshape(-1)
    gathered = _roi_gather(table, rois_p)
    pooled = gathered.reshape(PD, PH, PW, NR, C)
    return jnp.transpose(pooled, (3, 4, 0, 1, 2))


# speedup vs baseline: 9.7655x; 1.0071x over previous
import jax
import jax.numpy as jnp
import numpy as np
from jax import lax
from jax.experimental import pallas as pl
from jax.experimental.pallas import tpu as pltpu
from jax.experimental.pallas import tpu_sc as plsc

B, C, D, H, W = 2, 128, 32, 64, 64
DHW = D * H * W
HW = H * W
PD = PH = PW = 7
CELLS = PD * PH * PW
NR = 1000
NR_PAD = 1024
NGRP_FULL = NR // 16
CELLS_MAX = 22
CH = 40
NBUF = 12

_LIN_T = [np.float32(np.float32(k) / np.float32(6)) for k in range(6)]
_LIN_OMT = [np.float32(np.float32(1) - t) for t in _LIN_T]


def _grid_indices(a, b, hi):
    af = jnp.clip(a, 0.0, float(hi - 1))
    bf = jnp.clip(b, 0.0, float(hi - 1))
    out = []
    for k in range(7):
        if k == 6:
            g = bf
        else:
            g = af * _LIN_OMT[k] + bf * _LIN_T[k]
        out.append(jnp.clip(g.astype(jnp.int32), 0, hi - 1))
    return out


def _group_contribs(roi_v, g, lanes):
    base_addr = g * 128 + lanes * 8

    def fld(f):
        return plsc.load_gather(roi_v, [base_addr + f])

    bi = jnp.clip(fld(0).astype(jnp.int32), 0, B - 1)
    x1, y1, z1 = fld(1), fld(2), fld(3)
    x2, y2, z2 = fld(4), fld(5), fld(6)
    ix = _grid_indices(x1, x2, W)
    iy = _grid_indices(y1, y2, H)
    iz = _grid_indices(z1, z2, D)
    rowb = bi * DHW
    zc = [rowb + v * HW for v in iz]
    yc = [v * W for v in iy]
    return zc + yc + ix


def _roi_gather_body(table, rois, out, roi_v, tab_v, idx_v, bufs, gsems, wsems):
    s = lax.axis_index("s")
    c = lax.axis_index("c")
    base_cell = jnp.where(s < 7, s * 22, 21 * s + 7)
    ncells = jnp.where(s < 7, 22, 21)
    nch = jnp.where(c == 0, ncells * (1000 // CH), 0)

    lanes = lax.iota(jnp.int32, 16)

    @pl.when(c == 0)
    def _():
        pltpu.sync_copy(rois, roi_v)

        def phase1(g, carry):
            vecs = _group_contribs(roi_v, g, lanes)
            for k in range(21):
                tab_v[pl.ds(k * 1000 + g * 16, 16)] = vecs[k]
            return carry

        lax.fori_loop(0, NGRP_FULL, phase1, None)
        vecs = _group_contribs(roi_v, NGRP_FULL, lanes)
        tail_mask = lanes < 8
        tail_pos = NGRP_FULL * 16 + lanes
        for k in range(21):
            plsc.store_scatter(tab_v, [k * 1000 + tail_pos], vecs[k],
                               mask=tail_mask)

        addr = []
        for jj in range(CELLS_MAX):
            j = base_cell + jj
            k3 = j // 49
            k2 = (j // 7) % 7
            k1 = j % 7
            addr.append((k3 * 1000, (7 + k2) * 1000, (14 + k1) * 1000))

        def phase2(g, carry):
            g16 = g * 16
            for jj in range(CELLS_MAX):
                az, ay, ax = addr[jj]
                v = (tab_v[pl.ds(az + g16, 16)]
                     + tab_v[pl.ds(ay + g16, 16)]
                     + tab_v[pl.ds(ax + g16, 16)])
                idx_v[pl.ds(jj * 1000 + g16, 16)] = v
            return carry

        lax.fori_loop(0, NGRP_FULL, phase2, None)
        tp = NGRP_FULL * 16
        for jj in range(CELLS_MAX):
            az, ay, ax = addr[jj]
            v = (tab_v[pl.ds(az + tp, 16)]
                 + tab_v[pl.ds(ay + tp, 16)]
                 + tab_v[pl.ds(ax + tp, 16)])
            plsc.store_scatter(idx_v, [jj * 1000 + tp + lanes], v,
                               mask=tail_mask)

    out_base = base_cell * 1000

    def ring(r, _):
        for b in range(NBUF):
            t = NBUF * r + b

            @pl.when(t < nch)
            def _():
                @pl.when(t >= NBUF)
                def _():
                    pltpu.make_async_copy(
                        bufs[b], out.at[pl.ds(0, CH)], wsems[b]).wait()

                pltpu.async_copy(
                    table.at[idx_v.at[pl.ds(t * CH, CH)]], bufs[b], gsems[b])

            t2 = t - (NBUF - 1)
            b2 = (b + 1) % NBUF

            @pl.when(jnp.logical_and(t2 >= 0, t2 < nch))
            def _():
                pltpu.make_async_copy(
                    table.at[idx_v.at[pl.ds(0, CH)]], bufs[b2], gsems[b2]).wait()
                pltpu.async_copy(
                    bufs[b2], out.at[pl.ds(out_base + t2 * CH, CH)], wsems[b2])
        return _

    lax.fori_loop(0, (nch + 2 * (NBUF - 1)) // NBUF, ring, None)
    for b in range(NBUF):
        @pl.when(nch > b)
        def _():
            pltpu.make_async_copy(bufs[b], out.at[pl.ds(0, CH)], wsems[b]).wait()


def _body(table, rois, out, roi_v, tab_v, idx_v, *rest):
    _roi_gather_body(table, rois, out, roi_v, tab_v, idx_v,
                     rest[:NBUF], rest[NBUF:2 * NBUF], rest[2 * NBUF:])


_mesh = plsc.VectorSubcoreMesh(core_axis_name="c", subcore_axis_name="s")

_roi_gather = pl.kernel(
    _body,
    out_type=jax.ShapeDtypeStruct((CELLS * NR, C), jnp.float32),
    mesh=_mesh,
    scratch_types=[
        pltpu.VMEM((NR_PAD * 8,), jnp.float32),
        pltpu.VMEM((21 * 1000 + 8, ), jnp.int32),
        pltpu.VMEM((CELLS_MAX * 1000,), jnp.int32),
    ] + [pltpu.VMEM((CH, C), jnp.float32)] * NBUF
      + [pltpu.SemaphoreType.DMA] * (2 * NBUF),
    compiler_params=pltpu.CompilerParams(needs_layout_passes=False),
)


@jax.jit
def kernel(features, rois):
    table = jnp.transpose(features, (0, 2, 3, 4, 1)).reshape(B * DHW, C)
    rois_p = jnp.pad(rois, ((0, NR_PAD - NR), (0, 1))).reshape(-1)
    gathered = _roi_gather(table, rois_p)
    pooled = gathered.reshape(PD, PH, PW, NR, C)
    return jnp.transpose(pooled, (3, 4, 0, 1, 2))


# speedup vs baseline: 9.7748x; 1.0010x over previous
import jax
import jax.numpy as jnp
import numpy as np
from jax import lax
from jax.experimental import pallas as pl
from jax.experimental.pallas import tpu as pltpu
from jax.experimental.pallas import tpu_sc as plsc

B, C, D, H, W = 2, 128, 32, 64, 64
DHW = D * H * W
HW = H * W
PD = PH = PW = 7
CELLS = PD * PH * PW
NR = 1000
NR_PAD = 1024
NGRP_FULL = NR // 16
CELLS_MAX = 22
CH = 112
NBUF = 4

_LIN_T = [np.float32(np.float32(k) / np.float32(6)) for k in range(6)]
_LIN_OMT = [np.float32(np.float32(1) - t) for t in _LIN_T]


def _grid_indices(a, b, hi):
    af = jnp.clip(a, 0.0, float(hi - 1))
    bf = jnp.clip(b, 0.0, float(hi - 1))
    out = []
    for k in range(7):
        if k == 6:
            g = bf
        else:
            g = af * _LIN_OMT[k] + bf * _LIN_T[k]
        out.append(jnp.clip(g.astype(jnp.int32), 0, hi - 1))
    return out


def _group_contribs(roi_v, g, lanes):
    base_addr = g * 128 + lanes * 8

    def fld(f):
        return plsc.load_gather(roi_v, [base_addr + f])

    bi = jnp.clip(fld(0).astype(jnp.int32), 0, B - 1)
    x1, y1, z1 = fld(1), fld(2), fld(3)
    x2, y2, z2 = fld(4), fld(5), fld(6)
    ix = _grid_indices(x1, x2, W)
    iy = _grid_indices(y1, y2, H)
    iz = _grid_indices(z1, z2, D)
    rowb = bi * DHW
    zc = [rowb + v * HW for v in iz]
    yc = [v * W for v in iy]
    return zc + yc + ix


def _roi_gather_body(table, rois, out, roi_v, tab_v, idx_v, bufs, gsems, wsems,
                     tbufs):
    s = lax.axis_index("s")
    c = lax.axis_index("c")
    base_cell = jnp.where(s < 7, s * 22, 21 * s + 7)
    ncells = jnp.where(s < 7, 22, 21)
    nch = jnp.where(c == 0, ncells * 1000 // CH, 0)

    lanes = lax.iota(jnp.int32, 16)

    @pl.when(c == 0)
    def _():
        pltpu.sync_copy(rois, roi_v)

        def phase1(g, carry):
            vecs = _group_contribs(roi_v, g, lanes)
            for k in range(21):
                tab_v[pl.ds(k * 1000 + g * 16, 16)] = vecs[k]
            return carry

        lax.fori_loop(0, NGRP_FULL, phase1, None)
        vecs = _group_contribs(roi_v, NGRP_FULL, lanes)
        tail_mask = lanes < 8
        tail_pos = NGRP_FULL * 16 + lanes
        for k in range(21):
            plsc.store_scatter(tab_v, [k * 1000 + tail_pos], vecs[k],
                               mask=tail_mask)

        addr = []
        for jj in range(CELLS_MAX):
            j = base_cell + jj
            k3 = j // 49
            k2 = (j // 7) % 7
            k1 = j % 7
            addr.append((k3 * 1000, (7 + k2) * 1000, (14 + k1) * 1000))

        def phase2(g, carry):
            g16 = g * 16
            for jj in range(CELLS_MAX):
                az, ay, ax = addr[jj]
                v = (tab_v[pl.ds(az + g16, 16)]
                     + tab_v[pl.ds(ay + g16, 16)]
                     + tab_v[pl.ds(ax + g16, 16)])
                idx_v[pl.ds(jj * 1000 + g16, 16)] = v
            return carry

        lax.fori_loop(0, NGRP_FULL, phase2, None)
        tp = NGRP_FULL * 16
        for jj in range(CELLS_MAX):
            az, ay, ax = addr[jj]
            v = (tab_v[pl.ds(az + tp, 16)]
                 + tab_v[pl.ds(ay + tp, 16)]
                 + tab_v[pl.ds(ax + tp, 16)])
            plsc.store_scatter(idx_v, [jj * 1000 + tp + lanes], v,
                               mask=tail_mask)

    out_base = base_cell * 1000

    def ring(r, _):
        for b in range(NBUF):
            t = NBUF * r + b

            @pl.when(t < nch)
            def _():
                @pl.when(t >= NBUF)
                def _():
                    pltpu.make_async_copy(
                        bufs[b], out.at[pl.ds(0, CH)], wsems[b]).wait()

                pltpu.async_copy(
                    table.at[idx_v.at[pl.ds(t * CH, CH)]], bufs[b], gsems[b])

            t2 = t - (NBUF - 1)
            b2 = (b + 1) % NBUF

            @pl.when(jnp.logical_and(t2 >= 0, t2 < nch))
            def _():
                pltpu.make_async_copy(
                    table.at[idx_v.at[pl.ds(0, CH)]], bufs[b2], gsems[b2]).wait()
                pltpu.async_copy(
                    bufs[b2], out.at[pl.ds(out_base + t2 * CH, CH)], wsems[b2])
        return _

    lax.fori_loop(0, (nch + 2 * (NBUF - 1)) // NBUF, ring, None)
    for b in range(NBUF):
        @pl.when(nch > b)
        def _():
            pltpu.make_async_copy(bufs[b], out.at[pl.ds(0, CH)], wsems[b]).wait()

    @pl.when(c == 0)
    def _():
        toff = nch * CH
        for tbuf, pred in ((tbufs[0], s < 7), (tbufs[1], s >= 7)):
            @pl.when(pred)
            def _(tbuf=tbuf):
                sz = tbuf.shape[0]
                pltpu.async_copy(
                    table.at[idx_v.at[pl.ds(toff, sz)]], tbuf, gsems[0]).wait()
                pltpu.async_copy(
                    tbuf, out.at[pl.ds(out_base + toff, sz)], wsems[0])
                pltpu.make_async_copy(
                    tbuf, out.at[pl.ds(0, sz)], wsems[0]).wait()


def _body(table, rois, out, roi_v, tab_v, idx_v, *rest):
    _roi_gather_body(table, rois, out, roi_v, tab_v, idx_v,
                     rest[:NBUF], rest[NBUF:2 * NBUF], rest[2 * NBUF:3 * NBUF],
                     rest[3 * NBUF:])


_mesh = plsc.VectorSubcoreMesh(core_axis_name="c", subcore_axis_name="s")

_roi_gather = pl.kernel(
    _body,
    out_type=jax.ShapeDtypeStruct((CELLS * NR, C), jnp.float32),
    mesh=_mesh,
    scratch_types=[
        pltpu.VMEM((NR_PAD * 8,), jnp.float32),
        pltpu.VMEM((21 * 1000 + 8, ), jnp.int32),
        pltpu.VMEM((CELLS_MAX * 1000,), jnp.int32),
    ] + [pltpu.VMEM((CH, C), jnp.float32)] * NBUF
      + [pltpu.SemaphoreType.DMA] * (2 * NBUF)
      + [pltpu.VMEM((48, C), jnp.float32), pltpu.VMEM((56, C), jnp.float32)],
    compiler_params=pltpu.CompilerParams(needs_layout_passes=False),
)


@jax.jit
def kernel(features, rois):
    table = jnp.transpose(features, (0, 2, 3, 4, 1)).reshape(B * DHW, C)
    rois_p = jnp.pad(rois, ((0, NR_PAD - NR), (0, 1))).reshape(-1)
    gathered = _roi_gather(table, rois_p)
    pooled = gathered.reshape(PD, PH, PW, NR, C)
    return jnp.transpose(pooled, (3, 4, 0, 1, 2))


# speedup vs baseline: 10.0256x; 1.0257x over previous
import jax
import jax.numpy as jnp
import numpy as np
from jax import lax
from jax.experimental import pallas as pl
from jax.experimental.pallas import tpu as pltpu
from jax.experimental.pallas import tpu_sc as plsc

B, C, D, H, W = 2, 128, 32, 64, 64
DHW = D * H * W
HW = H * W
PD = PH = PW = 7
CELLS = PD * PH * PW
NR = 1000
NR_PAD = 1024
NGRP_FULL = NR // 16
CELLS_MAX = 22
CH = 64
NBUF = 8

_LIN_T = [np.float32(np.float32(k) / np.float32(6)) for k in range(6)]
_LIN_OMT = [np.float32(np.float32(1) - t) for t in _LIN_T]


def _grid_indices(a, b, hi):
    af = jnp.clip(a, 0.0, float(hi - 1))
    bf = jnp.clip(b, 0.0, float(hi - 1))
    out = []
    for k in range(7):
        if k == 6:
            g = bf
        else:
            g = af * _LIN_OMT[k] + bf * _LIN_T[k]
        out.append(jnp.clip(g.astype(jnp.int32), 0, hi - 1))
    return out


def _group_contribs(roi_v, g, lanes):
    base_addr = g * 128 + lanes * 8

    def fld(f):
        return plsc.load_gather(roi_v, [base_addr + f])

    bi = jnp.clip(fld(0).astype(jnp.int32), 0, B - 1)
    x1, y1, z1 = fld(1), fld(2), fld(3)
    x2, y2, z2 = fld(4), fld(5), fld(6)
    ix = _grid_indices(x1, x2, W)
    iy = _grid_indices(y1, y2, H)
    iz = _grid_indices(z1, z2, D)
    rowb = bi * DHW
    zc = [rowb + v * HW for v in iz]
    yc = [v * W for v in iy]
    return zc + yc + ix


def _roi_gather_body(table, rois, out, roi_v, tab_v, idx_v, bufs, gsems, wsems,
                     tbufs):
    s = lax.axis_index("s")
    c = lax.axis_index("c")
    base_cell = jnp.where(s < 7, s * 22, 21 * s + 7)
    ncells = jnp.where(s < 7, 22, 21)
    nch = jnp.where(c == 0, ncells * 1000 // CH, 0)

    lanes = lax.iota(jnp.int32, 16)

    @pl.when(c == 0)
    def _():
        pltpu.sync_copy(rois, roi_v)

        def phase1(g, carry):
            vecs = _group_contribs(roi_v, g, lanes)
            for k in range(21):
                tab_v[pl.ds(k * 1000 + g * 16, 16)] = vecs[k]
            return carry

        lax.fori_loop(0, NGRP_FULL, phase1, None)
        vecs = _group_contribs(roi_v, NGRP_FULL, lanes)
        tail_mask = lanes < 8
        tail_pos = NGRP_FULL * 16 + lanes
        for k in range(21):
            plsc.store_scatter(tab_v, [k * 1000 + tail_pos], vecs[k],
                               mask=tail_mask)

        addr = []
        for jj in range(CELLS_MAX):
            j = base_cell + jj
            k3 = j // 49
            k2 = (j // 7) % 7
            k1 = j % 7
            addr.append((k3 * 1000, (7 + k2) * 1000, (14 + k1) * 1000))

        def phase2(g, carry):
            g16 = g * 16
            for jj in range(CELLS_MAX):
                az, ay, ax = addr[jj]
                v = (tab_v[pl.ds(az + g16, 16)]
                     + tab_v[pl.ds(ay + g16, 16)]
                     + tab_v[pl.ds(ax + g16, 16)])
                idx_v[pl.ds(jj * 1000 + g16, 16)] = v
            return carry

        lax.fori_loop(0, NGRP_FULL, phase2, None)
        tp = NGRP_FULL * 16
        for jj in range(CELLS_MAX):
            az, ay, ax = addr[jj]
            v = (tab_v[pl.ds(az + tp, 16)]
                 + tab_v[pl.ds(ay + tp, 16)]
                 + tab_v[pl.ds(ax + tp, 16)])
            plsc.store_scatter(idx_v, [jj * 1000 + tp + lanes], v,
                               mask=tail_mask)

    out_base = base_cell * 1000

    def ring(r, _):
        for b in range(NBUF):
            t = NBUF * r + b

            @pl.when(t < nch)
            def _():
                @pl.when(t >= NBUF)
                def _():
                    pltpu.make_async_copy(
                        bufs[b], out.at[pl.ds(0, CH)], wsems[b]).wait()

                pltpu.async_copy(
                    table.at[idx_v.at[pl.ds(t * CH, CH)]], bufs[b], gsems[b])

            t2 = t - (NBUF - 1)
            b2 = (b + 1) % NBUF

            @pl.when(jnp.logical_and(t2 >= 0, t2 < nch))
            def _():
                pltpu.make_async_copy(
                    table.at[idx_v.at[pl.ds(0, CH)]], bufs[b2], gsems[b2]).wait()
                pltpu.async_copy(
                    bufs[b2], out.at[pl.ds(out_base + t2 * CH, CH)], wsems[b2])
        return _

    lax.fori_loop(0, (nch + 2 * (NBUF - 1)) // NBUF, ring, None)
    for b in range(NBUF):
        @pl.when(nch > b)
        def _():
            pltpu.make_async_copy(bufs[b], out.at[pl.ds(0, CH)], wsems[b]).wait()

    @pl.when(c == 0)
    def _():
        toff = nch * CH
        for tbuf, pred in ((tbufs[0], s < 7), (tbufs[1], s >= 7)):
            @pl.when(pred)
            def _(tbuf=tbuf):
                sz = tbuf.shape[0]
                pltpu.async_copy(
                    table.at[idx_v.at[pl.ds(toff, sz)]], tbuf, gsems[0]).wait()
                pltpu.async_copy(
                    tbuf, out.at[pl.ds(out_base + toff, sz)], wsems[0])
                pltpu.make_async_copy(
                    tbuf, out.at[pl.ds(0, sz)], wsems[0]).wait()


def _body(table, rois, out, roi_v, tab_v, idx_v, *rest):
    _roi_gather_body(table, rois, out, roi_v, tab_v, idx_v,
                     rest[:NBUF], rest[NBUF:2 * NBUF], rest[2 * NBUF:3 * NBUF],
                     rest[3 * NBUF:])


_mesh = plsc.VectorSubcoreMesh(core_axis_name="c", subcore_axis_name="s")

_roi_gather = pl.kernel(
    _body,
    out_type=jax.ShapeDtypeStruct((CELLS * NR, C), jnp.float32),
    mesh=_mesh,
    scratch_types=[
        pltpu.VMEM((NR_PAD * 8,), jnp.float32),
        pltpu.VMEM((21 * 1000 + 8, ), jnp.int32),
        pltpu.VMEM((CELLS_MAX * 1000,), jnp.int32),
    ] + [pltpu.VMEM((CH, C), jnp.float32)] * NBUF
      + [pltpu.SemaphoreType.DMA] * (2 * NBUF)
      + [pltpu.VMEM((48, C), jnp.float32), pltpu.VMEM((8, C), jnp.float32)],
    compiler_params=pltpu.CompilerParams(needs_layout_passes=False),
)


@jax.jit
def kernel(features, rois):
    table = jnp.transpose(features, (0, 2, 3, 4, 1)).reshape(B * DHW, C)
    rois_p = jnp.pad(rois, ((0, NR_PAD - NR), (0, 1))).reshape(-1)
    gathered = _roi_gather(table, rois_p)
    pooled = gathered.reshape(PD, PH, PW, NR, C)
    return jnp.transpose(pooled, (3, 4, 0, 1, 2))


# speedup vs baseline: 10.1713x; 1.0145x over previous
import jax
import jax.numpy as jnp
import numpy as np
from jax import lax
from jax.experimental import pallas as pl
from jax.experimental.pallas import tpu as pltpu
from jax.experimental.pallas import tpu_sc as plsc

B, C, D, H, W = 2, 128, 32, 64, 64
DHW = D * H * W
HW = H * W
PD = PH = PW = 7
CELLS = PD * PH * PW
NR = 1000
NR_PAD = 1024
NGRP_FULL = NR // 16
CELLS_MAX = 22
CH = 64
NBUF = 8

_LIN_T = [np.float32(np.float32(k) / np.float32(6)) for k in range(6)]
_LIN_OMT = [np.float32(np.float32(1) - t) for t in _LIN_T]


def _grid_indices(a, b, hi):
    af = jnp.clip(a, 0.0, float(hi - 1))
    bf = jnp.clip(b, 0.0, float(hi - 1))
    out = []
    for k in range(7):
        if k == 6:
            g = bf
        else:
            g = af * _LIN_OMT[k] + bf * _LIN_T[k]
        out.append(jnp.clip(g.astype(jnp.int32), 0, hi - 1))
    return out


def _group_contribs(roi_v, g, lanes):
    base_addr = g * 128 + lanes * 8

    def fld(f):
        return plsc.load_gather(roi_v, [base_addr + f])

    bi = jnp.clip(fld(0).astype(jnp.int32), 0, B - 1)
    x1, y1, z1 = fld(1), fld(2), fld(3)
    x2, y2, z2 = fld(4), fld(5), fld(6)
    ix = _grid_indices(x1, x2, W)
    iy = _grid_indices(y1, y2, H)
    iz = _grid_indices(z1, z2, D)
    rowb = bi * DHW
    zc = [rowb + v * HW for v in iz]
    yc = [v * W for v in iy]
    return zc + yc + ix


def _roi_gather_body(table, rois, out, roi_v, tab_v, idx_v, bufs, gsems, wsems,
                     tbufs):
    s = lax.axis_index("s")
    c = lax.axis_index("c")
    base_cell = jnp.where(s < 7, s * 22, 21 * s + 7)
    ncells = jnp.where(s < 7, 22, 21)
    nch = jnp.where(c == 0, ncells * 1000 // CH, 0)

    lanes = lax.iota(jnp.int32, 16)

    @pl.when(c == 0)
    def _():
        pltpu.sync_copy(rois, roi_v)

        def phase1(g, carry):
            vecs = _group_contribs(roi_v, g, lanes)
            for k in range(21):
                tab_v[pl.ds(k * 1000 + g * 16, 16)] = vecs[k]
            return carry

        lax.fori_loop(0, NGRP_FULL, phase1, None)
        vecs = _group_contribs(roi_v, NGRP_FULL, lanes)
        tail_mask = lanes < 8
        tail_pos = NGRP_FULL * 16 + lanes
        for k in range(21):
            plsc.store_scatter(tab_v, [k * 1000 + tail_pos], vecs[k],
                               mask=tail_mask)

        addr = []
        for jj in range(CELLS_MAX):
            j = base_cell + jj
            k3 = j // 49
            k2 = (j // 7) % 7
            k1 = j % 7
            addr.append((k3 * 1000, (7 + k2) * 1000, (14 + k1) * 1000))

        def phase2(g, carry):
            g16 = g * 16
            for jj in range(1, CELLS_MAX):
                az, ay, ax = addr[jj]
                v = (tab_v[pl.ds(az + g16, 16)]
                     + tab_v[pl.ds(ay + g16, 16)]
                     + tab_v[pl.ds(ax + g16, 16)])
                idx_v[pl.ds(jj * 1000 + g16, 16)] = v
            return carry

        tp = NGRP_FULL * 16

        def cell_idx(jj, g16):
            az, ay, ax = addr[jj]
            return (tab_v[pl.ds(az + g16, 16)]
                    + tab_v[pl.ds(ay + g16, 16)]
                    + tab_v[pl.ds(ax + g16, 16)])

        def phase2_cell0(g, carry):
            idx_v[pl.ds(g * 16, 16)] = cell_idx(0, g * 16)
            return carry

        lax.fori_loop(0, NGRP_FULL, phase2_cell0, None)
        plsc.store_scatter(idx_v, [tp + lanes], cell_idx(0, tp),
                           mask=tail_mask)
        for b in range(NBUF):
            pltpu.async_copy(
                table.at[idx_v.at[pl.ds(b * CH, CH)]], bufs[b], gsems[b])

        lax.fori_loop(0, NGRP_FULL, phase2, None)
        for jj in range(1, CELLS_MAX):
            v = cell_idx(jj, tp)
            plsc.store_scatter(idx_v, [jj * 1000 + tp + lanes], v,
                               mask=tail_mask)

    out_base = base_cell * 1000

    def ring(r, _):
        for b in range(NBUF):
            t = NBUF * r + b

            @pl.when(jnp.logical_and(t >= NBUF, t < nch))
            def _():
                pltpu.make_async_copy(
                    bufs[b], out.at[pl.ds(0, CH)], wsems[b]).wait()
                pltpu.async_copy(
                    table.at[idx_v.at[pl.ds(t * CH, CH)]], bufs[b], gsems[b])

            t2 = t - (NBUF - 1)
            b2 = (b + 1) % NBUF

            @pl.when(jnp.logical_and(t2 >= 0, t2 < nch))
            def _():
                pltpu.make_async_copy(
                    table.at[idx_v.at[pl.ds(0, CH)]], bufs[b2], gsems[b2]).wait()
                pltpu.async_copy(
                    bufs[b2], out.at[pl.ds(out_base + t2 * CH, CH)], wsems[b2])
        return _

    lax.fori_loop(0, (nch + 2 * (NBUF - 1)) // NBUF, ring, None)
    for b in range(NBUF):
        @pl.when(nch > b)
        def _():
            pltpu.make_async_copy(bufs[b], out.at[pl.ds(0, CH)], wsems[b]).wait()

    @pl.when(c == 0)
    def _():
        toff = nch * CH
        for tbuf, pred in ((tbufs[0], s < 7), (tbufs[1], s >= 7)):
            @pl.when(pred)
            def _(tbuf=tbuf):
                sz = tbuf.shape[0]
                pltpu.async_copy(
                    table.at[idx_v.at[pl.ds(toff, sz)]], tbuf, gsems[0]).wait()
                pltpu.async_copy(
                    tbuf, out.at[pl.ds(out_base + toff, sz)], wsems[0])
                pltpu.make_async_copy(
                    tbuf, out.at[pl.ds(0, sz)], wsems[0]).wait()


def _body(table, rois, out, roi_v, tab_v, idx_v, *rest):
    _roi_gather_body(table, rois, out, roi_v, tab_v, idx_v,
                     rest[:NBUF], rest[NBUF:2 * NBUF], rest[2 * NBUF:3 * NBUF],
                     rest[3 * NBUF:])


_mesh = plsc.VectorSubcoreMesh(core_axis_name="c", subcore_axis_name="s")

_roi_gather = pl.kernel(
    _body,
    out_type=jax.ShapeDtypeStruct((CELLS * NR, C), jnp.float32),
    mesh=_mesh,
    scratch_types=[
        pltpu.VMEM((NR_PAD * 8,), jnp.float32),
        pltpu.VMEM((21 * 1000 + 8, ), jnp.int32),
        pltpu.VMEM((CELLS_MAX * 1000,), jnp.int32),
    ] + [pltpu.VMEM((CH, C), jnp.float32)] * NBUF
      + [pltpu.SemaphoreType.DMA] * (2 * NBUF)
      + [pltpu.VMEM((48, C), jnp.float32), pltpu.VMEM((8, C), jnp.float32)],
    compiler_params=pltpu.CompilerParams(needs_layout_passes=False),
)


@jax.jit
def kernel(features, rois):
    table = jnp.transpose(features, (0, 2, 3, 4, 1)).reshape(B * DHW, C)
    rois_p = jnp.pad(rois, ((0, NR_PAD - NR), (0, 1))).reshape(-1)
    gathered = _roi_gather(table, rois_p)
    pooled = gathered.reshape(PD, PH, PW, NR, C)
    return jnp.transpose(pooled, (3, 4, 0, 1, 2))


# speedup vs baseline: 10.1926x; 1.0021x over previous
import jax
import jax.numpy as jnp
import numpy as np
from jax import lax
from jax.experimental import pallas as pl
from jax.experimental.pallas import tpu as pltpu
from jax.experimental.pallas import tpu_sc as plsc

B, C, D, H, W = 2, 128, 32, 64, 64
DHW = D * H * W
HW = H * W
PD = PH = PW = 7
CELLS = PD * PH * PW
NR = 1000
NR_PAD = 1024
NGRP_FULL = NR // 16
CELLS_MAX = 22
CH = 64
NBUF = 8

_LIN_T = [np.float32(np.float32(k) / np.float32(6)) for k in range(6)]
_LIN_OMT = [np.float32(np.float32(1) - t) for t in _LIN_T]


def _grid_indices(a, b, hi):
    af = jnp.clip(a, 0.0, float(hi - 1))
    bf = jnp.clip(b, 0.0, float(hi - 1))
    out = []
    for k in range(7):
        if k == 6:
            g = bf
        else:
            g = af * _LIN_OMT[k] + bf * _LIN_T[k]
        out.append(jnp.clip(g.astype(jnp.int32), 0, hi - 1))
    return out


def _group_contribs(roi_v, g, lanes):
    base_addr = g * 128 + lanes * 8

    def fld(f):
        return plsc.load_gather(roi_v, [base_addr + f])

    bi = jnp.clip(fld(0).astype(jnp.int32), 0, B - 1)
    x1, y1, z1 = fld(1), fld(2), fld(3)
    x2, y2, z2 = fld(4), fld(5), fld(6)
    ix = _grid_indices(x1, x2, W)
    iy = _grid_indices(y1, y2, H)
    iz = _grid_indices(z1, z2, D)
    rowb = bi * DHW
    zc = [rowb + v * HW for v in iz]
    yc = [v * W for v in iy]
    return zc + yc + ix


def _roi_gather_body(table, rois, out, roi_v, tab_v, idx_v, bufs, gsems, wsems,
                     tbufs):
    s = lax.axis_index("s")
    c = lax.axis_index("c")
    base_cell = jnp.where(s < 7, s * 22, 21 * s + 7)
    ncells = jnp.where(s < 7, 22, 21)
    nch = jnp.where(c == 0, ncells * 1000 // CH, 0)

    lanes = lax.iota(jnp.int32, 16)

    @pl.when(c == 0)
    def _():
        pltpu.sync_copy(rois, roi_v)

        def phase1(g, carry):
            vecs = _group_contribs(roi_v, g, lanes)
            for k in range(21):
                tab_v[pl.ds(k * 1000 + g * 16, 16)] = vecs[k]
            return carry

        tail_mask = lanes < 8
        NG_A = (NBUF * CH + 15) // 16
        lax.fori_loop(0, NG_A, phase1, None)

        addr = []
        for jj in range(CELLS_MAX):
            j = base_cell + jj
            k3 = j // 49
            k2 = (j // 7) % 7
            k1 = j % 7
            addr.append((k3 * 1000, (7 + k2) * 1000, (14 + k1) * 1000))

        def phase2(g, carry):
            g16 = g * 16
            for jj in range(1, CELLS_MAX):
                az, ay, ax = addr[jj]
                v = (tab_v[pl.ds(az + g16, 16)]
                     + tab_v[pl.ds(ay + g16, 16)]
                     + tab_v[pl.ds(ax + g16, 16)])
                idx_v[pl.ds(jj * 1000 + g16, 16)] = v
            return carry

        tp = NGRP_FULL * 16

        def cell_idx(jj, g16):
            az, ay, ax = addr[jj]
            return (tab_v[pl.ds(az + g16, 16)]
                    + tab_v[pl.ds(ay + g16, 16)]
                    + tab_v[pl.ds(ax + g16, 16)])

        def phase2_cell0(g, carry):
            idx_v[pl.ds(g * 16, 16)] = cell_idx(0, g * 16)
            return carry

        lax.fori_loop(0, NG_A, phase2_cell0, None)
        for b in range(NBUF):
            pltpu.async_copy(
                table.at[idx_v.at[pl.ds(b * CH, CH)]], bufs[b], gsems[b])

        lax.fori_loop(NG_A, NGRP_FULL, phase1, None)
        vecs = _group_contribs(roi_v, NGRP_FULL, lanes)
        tail_pos = NGRP_FULL * 16 + lanes
        for k in range(21):
            plsc.store_scatter(tab_v, [k * 1000 + tail_pos], vecs[k],
                               mask=tail_mask)

        lax.fori_loop(NG_A, NGRP_FULL, phase2_cell0, None)
        plsc.store_scatter(idx_v, [tp + lanes], cell_idx(0, tp),
                           mask=tail_mask)
        lax.fori_loop(0, NGRP_FULL, phase2, None)
        for jj in range(1, CELLS_MAX):
            v = cell_idx(jj, tp)
            plsc.store_scatter(idx_v, [jj * 1000 + tp + lanes], v,
                               mask=tail_mask)

    out_base = base_cell * 1000

    def ring(r, _):
        for b in range(NBUF):
            t = NBUF * r + b

            @pl.when(jnp.logical_and(t >= NBUF, t < nch))
            def _():
                pltpu.make_async_copy(
                    bufs[b], out.at[pl.ds(0, CH)], wsems[b]).wait()
                pltpu.async_copy(
                    table.at[idx_v.at[pl.ds(t * CH, CH)]], bufs[b], gsems[b])

            t2 = t - (NBUF - 1)
            b2 = (b + 1) % NBUF

            @pl.when(jnp.logical_and(t2 >= 0, t2 < nch))
            def _():
                pltpu.make_async_copy(
                    table.at[idx_v.at[pl.ds(0, CH)]], bufs[b2], gsems[b2]).wait()
                pltpu.async_copy(
                    bufs[b2], out.at[pl.ds(out_base + t2 * CH, CH)], wsems[b2])
        return _

    lax.fori_loop(0, (nch + 2 * (NBUF - 1)) // NBUF, ring, None)
    for b in range(NBUF):
        @pl.when(nch > b)
        def _():
            pltpu.make_async_copy(bufs[b], out.at[pl.ds(0, CH)], wsems[b]).wait()

    @pl.when(c == 0)
    def _():
        toff = nch * CH
        for tbuf, pred in ((tbufs[0], s < 7), (tbufs[1], s >= 7)):
            @pl.when(pred)
            def _(tbuf=tbuf):
                sz = tbuf.shape[0]
                pltpu.async_copy(
                    table.at[idx_v.at[pl.ds(toff, sz)]], tbuf, gsems[0]).wait()
                pltpu.async_copy(
                    tbuf, out.at[pl.ds(out_base + toff, sz)], wsems[0])
                pltpu.make_async_copy(
                    tbuf, out.at[pl.ds(0, sz)], wsems[0]).wait()


def _body(table, rois, out, roi_v, tab_v, idx_v, *rest):
    _roi_gather_body(table, rois, out, roi_v, tab_v, idx_v,
                     rest[:NBUF], rest[NBUF:2 * NBUF], rest[2 * NBUF:3 * NBUF],
                     rest[3 * NBUF:])


_mesh = plsc.VectorSubcoreMesh(core_axis_name="c", subcore_axis_name="s")

_roi_gather = pl.kernel(
    _body,
    out_type=jax.ShapeDtypeStruct((CELLS * NR, C), jnp.float32),
    mesh=_mesh,
    scratch_types=[
        pltpu.VMEM((NR_PAD * 8,), jnp.float32),
        pltpu.VMEM((21 * 1000 + 8, ), jnp.int32),
        pltpu.VMEM((CELLS_MAX * 1000,), jnp.int32),
    ] + [pltpu.VMEM((CH, C), jnp.float32)] * NBUF
      + [pltpu.SemaphoreType.DMA] * (2 * NBUF)
      + [pltpu.VMEM((48, C), jnp.float32), pltpu.VMEM((8, C), jnp.float32)],
    compiler_params=pltpu.CompilerParams(needs_layout_passes=False),
)


@jax.jit
def kernel(features, rois):
    table = jnp.transpose(features, (0, 2, 3, 4, 1)).reshape(B * DHW, C)
    rois_p = jnp.pad(rois, ((0, NR_PAD - NR), (0, 1))).reshape(-1)
    gathered = _roi_gather(table, rois_p)
    pooled = gathered.reshape(PD, PH, PW, NR, C)
    return jnp.transpose(pooled, (3, 4, 0, 1, 2))


# speedup vs baseline: 10.3311x; 1.0136x over previous
import jax
import jax.numpy as jnp
import numpy as np
from jax import lax
from jax.experimental import pallas as pl
from jax.experimental.pallas import tpu as pltpu
from jax.experimental.pallas import tpu_sc as plsc

B, C, D, H, W = 2, 128, 32, 64, 64
DHW = D * H * W
HW = H * W
PD = PH = PW = 7
CELLS = PD * PH * PW
NR = 1000
NR_PAD = 1024
NGRP_FULL = NR // 16
ROWS_W = 21440
CELLS_SPAN = 23
CH = 64
NBUF = 8
TAIL = ROWS_W - 40 - (ROWS_W - 40) // CH * CH

_LIN_T = [np.float32(np.float32(k) / np.float32(6)) for k in range(6)]
_LIN_OMT = [np.float32(np.float32(1) - t) for t in _LIN_T]


def _grid_indices(a, b, hi):
    af = jnp.clip(a, 0.0, float(hi - 1))
    bf = jnp.clip(b, 0.0, float(hi - 1))
    out = []
    for k in range(7):
        if k == 6:
            g = bf
        else:
            g = af * _LIN_OMT[k] + bf * _LIN_T[k]
        out.append(jnp.clip(g.astype(jnp.int32), 0, hi - 1))
    return out


def _group_contribs(roi_v, g, lanes):
    base_addr = g * 128 + lanes * 8

    def fld(f):
        return plsc.load_gather(roi_v, [base_addr + f])

    bi = jnp.clip(fld(0).astype(jnp.int32), 0, B - 1)
    x1, y1, z1 = fld(1), fld(2), fld(3)
    x2, y2, z2 = fld(4), fld(5), fld(6)
    ix = _grid_indices(x1, x2, W)
    iy = _grid_indices(y1, y2, H)
    iz = _grid_indices(z1, z2, D)
    rowb = bi * DHW
    zc = [rowb + v * HW for v in iz]
    yc = [v * W for v in iy]
    return zc + yc + ix


def _roi_gather_body(table, rois, out, roi_v, tab_v, idx_v, bufs, gsems, wsems,
                     tbuf):
    s = lax.axis_index("s")
    c = lax.axis_index("c")
    r0 = s * ROWS_W
    rlen = jnp.where(s < 15, ROWS_W, ROWS_W - 40)
    nch = jnp.where(c == 0, rlen // CH, 0)

    lanes = lax.iota(jnp.int32, 16)

    @pl.when(c == 0)
    def _():
        pltpu.sync_copy(rois, roi_v)

        def phase1(g, carry):
            vecs = _group_contribs(roi_v, g, lanes)
            for k in range(21):
                tab_v[pl.ds(k * 1000 + g * 16, 16)] = vecs[k]
            return carry

        lax.fori_loop(0, NGRP_FULL, phase1, None)
        vecs = _group_contribs(roi_v, NGRP_FULL, lanes)
        tail_mask = lanes < 8
        tail_pos = NGRP_FULL * 16 + lanes
        for k in range(21):
            plsc.store_scatter(tab_v, [k * 1000 + tail_pos], vecs[k],
                               mask=tail_mask)

        j0 = r0 // 1000

        def cell_phase2(jj):
            j = j0 + jj
            az = (j // 49) * 1000
            ay = (7 + (j // 7) % 7) * 1000
            ax = (14 + j % 7) * 1000
            jb = j * 1000
            n_lo = jnp.clip(r0 - jb, 0, 1000)
            n_hi = jnp.clip(r0 + rlen - jb, 0, 1000)
            lb = jb - r0

            def grp(g, carry):
                n = g * 16 + lanes
                v = (tab_v[pl.ds(az + g * 16, 16)]
                     + tab_v[pl.ds(ay + g * 16, 16)]
                     + tab_v[pl.ds(ax + g * 16, 16)])
                m = jnp.logical_and(n >= n_lo, n < n_hi)
                pos = jnp.clip(lb + n, 0, ROWS_W - 1)
                plsc.store_scatter(idx_v, [pos], v, mask=m)
                return carry

            lax.fori_loop(n_lo // 16, (n_hi + 15) // 16, grp, None)

        cell_phase2(0)
        cell_phase2(1)
        for b in range(NBUF):
            pltpu.async_copy(
                table.at[idx_v.at[pl.ds(b * CH, CH)]], bufs[b], gsems[b])
        for jj in range(2, CELLS_SPAN):
            cell_phase2(jj)

    def ring(r, _):
        for b in range(NBUF):
            t = NBUF * r + b

            @pl.when(jnp.logical_and(t >= NBUF, t < nch))
            def _():
                pltpu.make_async_copy(
                    bufs[b], out.at[pl.ds(0, CH)], wsems[b]).wait()
                pltpu.async_copy(
                    table.at[idx_v.at[pl.ds(t * CH, CH)]], bufs[b], gsems[b])

            t2 = t - (NBUF - 1)
            b2 = (b + 1) % NBUF

            @pl.when(jnp.logical_and(t2 >= 0, t2 < nch))
            def _():
                pltpu.make_async_copy(
                    table.at[idx_v.at[pl.ds(0, CH)]], bufs[b2], gsems[b2]).wait()
                pltpu.async_copy(
                    bufs[b2], out.at[pl.ds(r0 + t2 * CH, CH)], wsems[b2])
        return _

    lax.fori_loop(0, (nch + 2 * (NBUF - 1)) // NBUF, ring, None)
    for b in range(NBUF):
        @pl.when(nch > b)
        def _():
            pltpu.make_async_copy(bufs[b], out.at[pl.ds(0, CH)], wsems[b]).wait()

    @pl.when(jnp.logical_and(c == 0, s == 15))
    def _():
        toff = nch * CH
        pltpu.async_copy(
            table.at[idx_v.at[pl.ds(toff, TAIL)]], tbuf, gsems[0]).wait()
        pltpu.async_copy(tbuf, out.at[pl.ds(r0 + toff, TAIL)], wsems[0])
        pltpu.make_async_copy(tbuf, out.at[pl.ds(0, TAIL)], wsems[0]).wait()


def _body(table, rois, out, roi_v, tab_v, idx_v, *rest):
    _roi_gather_body(table, rois, out, roi_v, tab_v, idx_v,
                     rest[:NBUF], rest[NBUF:2 * NBUF], rest[2 * NBUF:3 * NBUF],
                     rest[3 * NBUF])


_mesh = plsc.VectorSubcoreMesh(core_axis_name="c", subcore_axis_name="s")

_roi_gather = pl.kernel(
    _body,
    out_type=jax.ShapeDtypeStruct((CELLS * NR, C), jnp.float32),
    mesh=_mesh,
    scratch_types=[
        pltpu.VMEM((NR_PAD * 8,), jnp.float32),
        pltpu.VMEM((21 * 1000 + 8,), jnp.int32),
        pltpu.VMEM((ROWS_W,), jnp.int32),
    ] + [pltpu.VMEM((CH, C), jnp.float32)] * NBUF
      + [pltpu.SemaphoreType.DMA] * (2 * NBUF)
      + [pltpu.VMEM((TAIL, C), jnp.float32)],
    compiler_params=pltpu.CompilerParams(needs_layout_passes=False),
)


@jax.jit
def kernel(features, rois):
    table = jnp.transpose(features, (0, 2, 3, 4, 1)).reshape(B * DHW, C)
    rois_p = jnp.pad(rois, ((0, NR_PAD - NR), (0, 1))).reshape(-1)
    gathered = _roi_gather(table, rois_p)
    pooled = gathered.reshape(PD, PH, PW, NR, C)
    return jnp.transpose(pooled, (3, 4, 0, 1, 2))


# speedup vs baseline: 10.3598x; 1.0028x over previous
import jax
import jax.numpy as jnp
import numpy as np
from jax import lax
from jax.experimental import pallas as pl
from jax.experimental.pallas import tpu as pltpu
from jax.experimental.pallas import tpu_sc as plsc

B, C, D, H, W = 2, 128, 32, 64, 64
DHW = D * H * W
HW = H * W
PD = PH = PW = 7
CELLS = PD * PH * PW
NR = 1000
NR_PAD = 1024
NGRP_FULL = NR // 16
ROWS_W = 21440
CELLS_SPAN = 23
CH = 64
NBUF = 9
TAIL = ROWS_W - 40 - (ROWS_W - 40) // CH * CH

_LIN_T = [np.float32(np.float32(k) / np.float32(6)) for k in range(6)]
_LIN_OMT = [np.float32(np.float32(1) - t) for t in _LIN_T]


def _grid_indices(a, b, hi):
    af = jnp.clip(a, 0.0, float(hi - 1))
    bf = jnp.clip(b, 0.0, float(hi - 1))
    out = []
    for k in range(7):
        if k == 6:
            g = bf
        else:
            g = af * _LIN_OMT[k] + bf * _LIN_T[k]
        out.append(jnp.clip(g.astype(jnp.int32), 0, hi - 1))
    return out


def _group_contribs(roi_v, g, lanes):
    base_addr = g * 128 + lanes * 8

    def fld(f):
        return plsc.load_gather(roi_v, [base_addr + f])

    bi = jnp.clip(fld(0).astype(jnp.int32), 0, B - 1)
    x1, y1, z1 = fld(1), fld(2), fld(3)
    x2, y2, z2 = fld(4), fld(5), fld(6)
    ix = _grid_indices(x1, x2, W)
    iy = _grid_indices(y1, y2, H)
    iz = _grid_indices(z1, z2, D)
    rowb = bi * DHW
    zc = [rowb + v * HW for v in iz]
    yc = [v * W for v in iy]
    return zc + yc + ix


def _roi_gather_body(table, rois, out, roi_v, tab_v, idx_v, bufs, gsems, wsems,
                     tbuf):
    s = lax.axis_index("s")
    c = lax.axis_index("c")
    r0 = s * ROWS_W
    rlen = jnp.where(s < 15, ROWS_W, ROWS_W - 40)
    nch = jnp.where(c == 0, rlen // CH, 0)

    lanes = lax.iota(jnp.int32, 16)

    @pl.when(c == 0)
    def _():
        pltpu.sync_copy(rois, roi_v)

        def phase1(g, carry):
            vecs = _group_contribs(roi_v, g, lanes)
            for k in range(21):
                tab_v[pl.ds(k * 1000 + g * 16, 16)] = vecs[k]
            return carry

        lax.fori_loop(0, NGRP_FULL, phase1, None)
        vecs = _group_contribs(roi_v, NGRP_FULL, lanes)
        tail_mask = lanes < 8
        tail_pos = NGRP_FULL * 16 + lanes
        for k in range(21):
            plsc.store_scatter(tab_v, [k * 1000 + tail_pos], vecs[k],
                               mask=tail_mask)

        j0 = r0 // 1000

        def cell_phase2(jj):
            j = j0 + jj
            az = (j // 49) * 1000
            ay = (7 + (j // 7) % 7) * 1000
            ax = (14 + j % 7) * 1000
            jb = j * 1000
            n_lo = jnp.clip(r0 - jb, 0, 1000)
            n_hi = jnp.clip(r0 + rlen - jb, 0, 1000)
            lb = jb - r0

            def grp(g, carry):
                n = g * 16 + lanes
                v = (tab_v[pl.ds(az + g * 16, 16)]
                     + tab_v[pl.ds(ay + g * 16, 16)]
                     + tab_v[pl.ds(ax + g * 16, 16)])
                m = jnp.logical_and(n >= n_lo, n < n_hi)
                pos = jnp.clip(lb + n, 0, ROWS_W - 1)
                plsc.store_scatter(idx_v, [pos], v, mask=m)
                return carry

            lax.fori_loop(n_lo // 16, (n_hi + 15) // 16, grp, None)

        cell_phase2(0)
        cell_phase2(1)
        for b in range(NBUF):
            pltpu.async_copy(
                table.at[idx_v.at[pl.ds(b * CH, CH)]], bufs[b], gsems[b])
        for jj in range(2, CELLS_SPAN):
            cell_phase2(jj)

    def ring(r, _):
        for b in range(NBUF):
            t = NBUF * r + b

            @pl.when(jnp.logical_and(t >= NBUF, t < nch))
            def _():
                pltpu.make_async_copy(
                    bufs[b], out.at[pl.ds(0, CH)], wsems[b]).wait()
                pltpu.async_copy(
                    table.at[idx_v.at[pl.ds(t * CH, CH)]], bufs[b], gsems[b])

            t2 = t - (NBUF - 1)
            b2 = (b + 1) % NBUF

            @pl.when(jnp.logical_and(t2 >= 0, t2 < nch))
            def _():
                pltpu.make_async_copy(
                    table.at[idx_v.at[pl.ds(0, CH)]], bufs[b2], gsems[b2]).wait()
                pltpu.async_copy(
                    bufs[b2], out.at[pl.ds(r0 + t2 * CH, CH)], wsems[b2])
        return _

    lax.fori_loop(0, (nch + 2 * (NBUF - 1)) // NBUF, ring, None)
    for b in range(NBUF):
        @pl.when(nch > b)
        def _():
            pltpu.make_async_copy(bufs[b], out.at[pl.ds(0, CH)], wsems[b]).wait()

    @pl.when(jnp.logical_and(c == 0, s == 15))
    def _():
        toff = nch * CH
        pltpu.async_copy(
            table.at[idx_v.at[pl.ds(toff, TAIL)]], tbuf, gsems[0]).wait()
        pltpu.async_copy(tbuf, out.at[pl.ds(r0 + toff, TAIL)], wsems[0])
        pltpu.make_async_copy(tbuf, out.at[pl.ds(0, TAIL)], wsems[0]).wait()


def _body(table, rois, out, roi_v, tab_v, idx_v, *rest):
    _roi_gather_body(table, rois, out, roi_v, tab_v, idx_v,
                     rest[:NBUF], rest[NBUF:2 * NBUF], rest[2 * NBUF:3 * NBUF],
                     rest[3 * NBUF])


_mesh = plsc.VectorSubcoreMesh(core_axis_name="c", subcore_axis_name="s")

_roi_gather = pl.kernel(
    _body,
    out_type=jax.ShapeDtypeStruct((CELLS * NR, C), jnp.float32),
    mesh=_mesh,
    scratch_types=[
        pltpu.VMEM((NR_PAD * 8,), jnp.float32),
        pltpu.VMEM((21 * 1000 + 8,), jnp.int32),
        pltpu.VMEM((ROWS_W,), jnp.int32),
    ] + [pltpu.VMEM((CH, C), jnp.float32)] * NBUF
      + [pltpu.SemaphoreType.DMA] * (2 * NBUF)
      + [pltpu.VMEM((TAIL, C), jnp.float32)],
    compiler_params=pltpu.CompilerParams(needs_layout_passes=False),
)


@jax.jit
def kernel(features, rois):
    table = jnp.transpose(features, (0, 2, 3, 4, 1)).reshape(B * DHW, C)
    rois_p = jnp.pad(rois, ((0, NR_PAD - NR), (0, 1))).reshape(-1)
    gathered = _roi_gather(table, rois_p)
    pooled = gathered.reshape(PD, PH, PW, NR, C)
    return jnp.transpose(pooled, (3, 4, 0, 1, 2))
